# Initial kernel scaffold; baseline (speedup 1.0000x reference)
import functools

import jax
import jax.numpy as jnp
from jax import lax
from jax.experimental import pallas as pl
from jax.experimental.pallas import tpu as pltpu
from jax.experimental.pallas import tpu_sc as plsc

N = 10000
E = 320000
F = 128
ALPHA = 0.2
G = 80
EB = E // G
NW = 32
GPT = EB // NW
N_PAD = 10240
NPT = N_PAD // 16

_sc_mesh = plsc.VectorSubcoreMesh(core_axis_name="c", subcore_axis_name="s")


def _proj_body(x_ref, w_ref, a_ref, h_ref, ha_ref):
    h = jnp.dot(x_ref[...], w_ref[...], preferred_element_type=jnp.float32)
    h_ref[...] = h
    ha_ref[...] = jnp.dot(h, a_ref[...], preferred_element_type=jnp.float32)


def _project(x, W, attn_pad):
    blk = 1000
    return pl.pallas_call(
        _proj_body,
        grid=(N // blk,),
        in_specs=[
            pl.BlockSpec((blk, F), lambda i: (i, 0)),
            pl.BlockSpec((F, F), lambda i: (0, 0)),
            pl.BlockSpec((F, F), lambda i: (0, 0)),
        ],
        out_specs=[
            pl.BlockSpec((blk, F), lambda i: (i, 0)),
            pl.BlockSpec((blk, F), lambda i: (i, 0)),
        ],
        out_shape=[
            jax.ShapeDtypeStruct((N, F), jnp.float32),
            jax.ShapeDtypeStruct((N, F), jnp.float32),
        ],
    )(x, W, attn_pad)


@functools.partial(
    pl.kernel,
    out_type=[
        jax.ShapeDtypeStruct((EB, G), jnp.float32),
        jax.ShapeDtypeStruct((2, N_PAD), jnp.float32),
    ],
    mesh=_sc_mesh,
    scratch_types=[
        pltpu.VMEM((N,), jnp.float32),
        pltpu.VMEM((N,), jnp.float32),
        pltpu.VMEM((GPT, G), jnp.int32),
        pltpu.VMEM((GPT, G), jnp.int32),
        pltpu.VMEM((GPT, G), jnp.float32),
        pltpu.VMEM((NPT,), jnp.float32),
        pltpu.VMEM_SHARED((N_PAD,), jnp.float32),
    ],
)
def _attn_kernel(asrc_hbm, adst_hbm, src_hbm, dst_hbm, ev_hbm, ssum_hbm,
                 asrc_v, adst_v, src_v, dst_v, ev_v, obuf, ssum_sh):
    c = lax.axis_index("c")
    s = lax.axis_index("s")
    wid = c * 16 + s
    rbase = wid * GPT
    pltpu.sync_copy(asrc_hbm, asrc_v)
    pltpu.sync_copy(adst_hbm, adst_v)
    pltpu.sync_copy(src_hbm.at[pl.ds(rbase, GPT)], src_v)
    pltpu.sync_copy(dst_hbm.at[pl.ds(rbase, GPT)], dst_v)

    def _zero(i, _):
        obuf[pl.ds(i * 16, 16)] = jnp.zeros((16,), jnp.float32)
        return 0
    lax.fori_loop(0, NPT // 16, _zero, 0)
    pltpu.sync_copy(obuf, ssum_sh.at[pl.ds(s * NPT, NPT)])
    plsc.subcore_barrier()

    def _group(g, _):
        def _sub(k, _):
            si = src_v[g, pl.ds(k * 16, 16)]
            di = dst_v[g, pl.ds(k * 16, 16)]
            v = plsc.load_gather(asrc_v, [si]) + plsc.load_gather(adst_v, [di])
            val = jnp.where(v > 0, -v, (-ALPHA) * v)
            ev_v[g, pl.ds(k * 16, 16)] = jnp.exp(val)
            return 0
        lax.fori_loop(0, G // 16, _sub, 0)
        pltpu.sync_copy(ev_v.at[g], ssum_sh.at[src_v.at[g]], add=True)
        return 0
    lax.fori_loop(0, GPT, _group, 0)

    pltpu.sync_copy(ev_v, ev_hbm.at[pl.ds(rbase, GPT)])
    plsc.subcore_barrier()
    pltpu.sync_copy(ssum_sh.at[pl.ds(s * NPT, NPT)], obuf)
    pltpu.sync_copy(obuf, ssum_hbm.at[c, pl.ds(s * NPT, NPT)])


@functools.partial(
    pl.kernel,
    out_type=jax.ShapeDtypeStruct((2, N_PAD, F), jnp.float32),
    mesh=_sc_mesh,
    scratch_types=[
        pltpu.VMEM((GPT, G), jnp.int32),
        pltpu.VMEM((GPT, G), jnp.int32),
        pltpu.VMEM((GPT, G), jnp.float32),
        pltpu.VMEM((2, N_PAD), jnp.float32),
        pltpu.VMEM((N_PAD,), jnp.float32),
        pltpu.VMEM((G, F), jnp.float32),
        pltpu.VMEM_SHARED((N_PAD, F), jnp.float32),
        pltpu.SemaphoreType.DMA,
    ],
)
def _spmm_kernel(h_hbm, src_hbm, dst_hbm, ev_hbm, ssump_hbm, out_hbm,
                 src_v, dst_v, w_v, ssump_v, ssum_v, rows_v, acc_sh, sem):
    c = lax.axis_index("c")
    s = lax.axis_index("s")
    wid = c * 16 + s
    rbase = wid * GPT
    pltpu.sync_copy(src_hbm.at[pl.ds(rbase, GPT)], src_v)
    pltpu.sync_copy(dst_hbm.at[pl.ds(rbase, GPT)], dst_v)
    pltpu.sync_copy(ev_hbm.at[pl.ds(rbase, GPT)], w_v)
    pltpu.sync_copy(ssump_hbm, ssump_v)

    def _comb(i, _):
        ssum_v[pl.ds(i * 16, 16)] = (
            ssump_v[0, pl.ds(i * 16, 16)] + ssump_v[1, pl.ds(i * 16, 16)])
        return 0
    lax.fori_loop(0, N_PAD // 16, _comb, 0)

    def _wgrp(g, _):
        def _sub(k, _):
            si = src_v[g, pl.ds(k * 16, 16)]
            ssv = plsc.load_gather(ssum_v, [si])
            w_v[g, pl.ds(k * 16, 16)] = w_v[g, pl.ds(k * 16, 16)] / (ssv + 1e-16)
            return 0
        lax.fori_loop(0, G // 16, _sub, 0)
        return 0
    lax.fori_loop(0, GPT, _wgrp, 0)

    def _zrow(i, _):
        def _zc(k, _):
            rows_v[i, pl.ds(k * 16, 16)] = jnp.zeros((16,), jnp.float32)
            return 0
        lax.fori_loop(0, F // 16, _zc, 0)
        return 0
    lax.fori_loop(0, G, _zrow, 0)

    def _zout(j, _):
        pltpu.sync_copy(rows_v, acc_sh.at[pl.ds(s * NPT + j * G, G)])
        return 0
    lax.fori_loop(0, NPT // G, _zout, 0)
    plsc.subcore_barrier()

    def _group(g, _):
        pltpu.async_copy(h_hbm.at[dst_v.at[g]], rows_v, sem).wait()

        def _edge(e, _):
            gb = jnp.full((16,), g, jnp.int32)
            eb = jnp.full((16,), e, jnp.int32)
            wb = plsc.load_gather(w_v, [gb, eb])

            def _chunk(k, _):
                rows_v[e, pl.ds(k * 16, 16)] = rows_v[e, pl.ds(k * 16, 16)] * wb
                return 0
            lax.fori_loop(0, F // 16, _chunk, 0)
            return 0
        lax.fori_loop(0, G, _edge, 0)
        pltpu.sync_copy(rows_v, acc_sh.at[src_v.at[g]], add=True)
        return 0
    lax.fori_loop(0, GPT, _group, 0)
    plsc.subcore_barrier()

    def _wout(j, _):
        pltpu.sync_copy(acc_sh.at[pl.ds(s * NPT + j * G, G)], rows_v)
        pltpu.sync_copy(rows_v, out_hbm.at[c, pl.ds(s * NPT + j * G, G)])
        return 0
    lax.fori_loop(0, NPT // G, _wout, 0)


def _elu_body(p0_ref, p1_ref, o_ref):
    hp = p0_ref[...] + p1_ref[...]
    o_ref[...] = jnp.where(hp > 0, hp, jnp.exp(jnp.minimum(hp, 0.0)) - 1.0)


def _elu_sum(p0, p1):
    blk = 1000
    return pl.pallas_call(
        _elu_body,
        grid=(N // blk,),
        in_specs=[
            pl.BlockSpec((blk, F), lambda i: (i, 0)),
            pl.BlockSpec((blk, F), lambda i: (i, 0)),
        ],
        out_specs=pl.BlockSpec((blk, F), lambda i: (i, 0)),
        out_shape=jax.ShapeDtypeStruct((N, F), jnp.float32),
    )(p0, p1)


def kernel(x, edge, W, attn):
    src2d = edge[0].astype(jnp.int32).reshape(EB, G)
    dst2d = edge[1].astype(jnp.int32).reshape(EB, G)
    attn_pad = jnp.zeros((F, F), jnp.float32)
    attn_pad = attn_pad.at[:, 0].set(attn[:F]).at[:, 1].set(attn[F:])
    h, ha = _project(x.astype(jnp.float32), W.astype(jnp.float32), attn_pad)
    a_src = ha[:, 0]
    a_dst = ha[:, 1]
    ev2d, ssum_p = _attn_kernel(a_src, a_dst, src2d, dst2d)
    out_p = _spmm_kernel(h, src2d, dst2d, ev2d, ssum_p)
    return _elu_sum(out_p[0, :N], out_p[1, :N])


# speedup vs baseline: 13.5760x; 13.5760x over previous
import functools

import jax
import jax.numpy as jnp
from jax import lax
from jax.experimental import pallas as pl
from jax.experimental.pallas import tpu as pltpu
from jax.experimental.pallas import tpu_sc as plsc

N = 10000
E = 320000
F = 128
ALPHA = 0.2
G = 80
EB = E // G
NW = 32
GPT = EB // NW
N_PAD = 10240
NPT = N_PAD // 16
GPB = EB // 16
FH = F // 2

_sc_mesh = plsc.VectorSubcoreMesh(core_axis_name="c", subcore_axis_name="s")
_sc_params = pltpu.CompilerParams(needs_layout_passes=False,
                                  use_tc_tiling_on_sc=False)


def _proj_body(x_ref, wb_ref, w_ref, a_ref, h2_ref, ha_ref):
    h2_ref[...] = jnp.dot(x_ref[...], wb_ref[0],
                          preferred_element_type=jnp.float32)
    wa = jnp.dot(w_ref[...], a_ref[...], preferred_element_type=jnp.float32)
    ha_ref[...] = jnp.dot(x_ref[...], wa, preferred_element_type=jnp.float32)


def _project(x, W, attn_pad):
    blk = 1000
    nb = N // blk
    return pl.pallas_call(
        _proj_body,
        grid=(nb, 2),
        in_specs=[
            pl.BlockSpec((blk, F), lambda i, j: (i, 0)),
            pl.BlockSpec((1, F, FH), lambda i, j: (j, 0, 0)),
            pl.BlockSpec((F, F), lambda i, j: (0, 0)),
            pl.BlockSpec((F, F), lambda i, j: (0, 0)),
        ],
        out_specs=[
            pl.BlockSpec((blk, FH), lambda i, j: (j * nb + i, 0)),
            pl.BlockSpec((blk, F), lambda i, j: (i, 0)),
        ],
        out_shape=[
            jax.ShapeDtypeStruct((2 * N, FH), jnp.float32),
            jax.ShapeDtypeStruct((N, F), jnp.float32),
        ],
    )(x, W.reshape(F, 2, FH).transpose(1, 0, 2), W, attn_pad)


@functools.partial(
    pl.kernel,
    out_type=[
        jax.ShapeDtypeStruct((NW, GPT, G), jnp.float32),
        jax.ShapeDtypeStruct((2, 1, N_PAD), jnp.float32),
    ],
    mesh=_sc_mesh,
    scratch_types=[
        pltpu.VMEM((N,), jnp.float32),
        pltpu.VMEM((N,), jnp.float32),
        pltpu.VMEM((GPT, G), jnp.int32),
        pltpu.VMEM((GPT, G), jnp.int32),
        pltpu.VMEM((GPT, G), jnp.float32),
        pltpu.VMEM((NPT,), jnp.float32),
        pltpu.VMEM_SHARED((N_PAD,), jnp.float32),
    ],
    compiler_params=_sc_params,
)
def _attn_kernel(asrc_hbm, adst_hbm, src_hbm, dst_hbm, ev_hbm, ssum_hbm,
                 asrc_v, adst_v, src_v, dst_v, ev_v, obuf, ssum_sh):
    c = lax.axis_index("c")
    s = lax.axis_index("s")
    wid = c * 16 + s
    pltpu.sync_copy(asrc_hbm, asrc_v)
    pltpu.sync_copy(adst_hbm, adst_v)
    pltpu.sync_copy(src_hbm.at[wid], src_v)
    pltpu.sync_copy(dst_hbm.at[wid], dst_v)

    def _zero(i, _):
        obuf[pl.ds(i * 16, 16)] = jnp.zeros((16,), jnp.float32)
        return 0
    lax.fori_loop(0, NPT // 16, _zero, 0)
    pltpu.sync_copy(obuf, ssum_sh.at[pl.ds(s * NPT, NPT)])
    plsc.subcore_barrier()

    def _group(g, _):
        def _sub(k, _):
            si = src_v[g, pl.ds(k * 16, 16)]
            di = dst_v[g, pl.ds(k * 16, 16)]
            v = plsc.load_gather(asrc_v, [si]) + plsc.load_gather(adst_v, [di])
            val = jnp.where(v > 0, -v, (-ALPHA) * v)
            ev_v[g, pl.ds(k * 16, 16)] = jnp.exp(val)
            return 0
        lax.fori_loop(0, G // 16, _sub, 0)
        pltpu.sync_copy(ev_v.at[g], ssum_sh.at[src_v.at[g]], add=True)
        return 0
    lax.fori_loop(0, GPT, _group, 0)

    pltpu.sync_copy(ev_v, ev_hbm.at[wid])
    plsc.subcore_barrier()
    pltpu.sync_copy(ssum_sh.at[pl.ds(s * NPT, NPT)], obuf)
    pltpu.sync_copy(obuf, ssum_hbm.at[c, 0, pl.ds(s * NPT, NPT)])


def _sum2_body(p_ref, o_ref):
    o_ref[...] = p_ref[0] + p_ref[1]


def _sum_partials(p):
    return pl.pallas_call(
        _sum2_body,
        in_specs=[pl.BlockSpec((2, 1, N_PAD), lambda: (0, 0, 0))],
        out_specs=pl.BlockSpec((1, N_PAD), lambda: (0, 0)),
        out_shape=jax.ShapeDtypeStruct((1, N_PAD), jnp.float32),
    )(p)


@functools.partial(
    pl.kernel,
    out_type=jax.ShapeDtypeStruct((2, N_PAD, FH), jnp.float32),
    mesh=_sc_mesh,
    scratch_types=[
        pltpu.VMEM((GPB, G), jnp.int32),
        pltpu.VMEM((GPB, G), jnp.int32),
        pltpu.VMEM((GPB, G), jnp.float32),
        pltpu.VMEM((N_PAD,), jnp.float32),
        pltpu.VMEM((G, FH), jnp.float32),
        pltpu.VMEM_SHARED((N_PAD, FH), jnp.float32),
        pltpu.SemaphoreType.DMA,
    ],
    compiler_params=_sc_params,
)
def _spmm_kernel(h2_hbm, src_hbm, dst_hbm, ev_hbm, ssum_hbm, out_hbm,
                 src_v, idx_v, w_v, ssum_v, rows_v, acc_sh, sem):
    c = lax.axis_index("c")
    s = lax.axis_index("s")
    pltpu.sync_copy(src_hbm.at[s], src_v)
    pltpu.sync_copy(dst_hbm.at[s], idx_v)
    pltpu.sync_copy(ev_hbm.at[s], w_v)
    pltpu.sync_copy(ssum_hbm.at[0], ssum_v)

    cn = jnp.full((16,), c * N, jnp.int32)

    def _wgrp(g, _):
        def _sub(k, _):
            si = src_v[g, pl.ds(k * 16, 16)]
            ssv = plsc.load_gather(ssum_v, [si])
            w_v[g, pl.ds(k * 16, 16)] = w_v[g, pl.ds(k * 16, 16)] / (ssv + 1e-16)
            idx_v[g, pl.ds(k * 16, 16)] = idx_v[g, pl.ds(k * 16, 16)] + cn
            return 0
        lax.fori_loop(0, G // 16, _sub, 0)
        return 0
    lax.fori_loop(0, GPB, _wgrp, 0)

    def _zrow(i, _):
        def _zc(k, _):
            rows_v[i, pl.ds(k * 16, 16)] = jnp.zeros((16,), jnp.float32)
            return 0
        lax.fori_loop(0, FH // 16, _zc, 0)
        return 0
    lax.fori_loop(0, G, _zrow, 0)

    def _zout(j, _):
        pltpu.sync_copy(rows_v, acc_sh.at[pl.ds(s * NPT + j * G, G)])
        return 0
    lax.fori_loop(0, NPT // G, _zout, 0)
    plsc.subcore_barrier()

    def _group(g, _):
        pltpu.async_copy(h2_hbm.at[idx_v.at[g]], rows_v, sem).wait()

        def _edge(e, _):
            gb = jnp.full((16,), g, jnp.int32)
            eb = jnp.full((16,), e, jnp.int32)
            wb = plsc.load_gather(w_v, [gb, eb])

            def _chunk(k, _):
                rows_v[e, pl.ds(k * 16, 16)] = rows_v[e, pl.ds(k * 16, 16)] * wb
                return 0
            lax.fori_loop(0, FH // 16, _chunk, 0)
            return 0
        lax.fori_loop(0, G, _edge, 0)
        pltpu.sync_copy(rows_v, acc_sh.at[src_v.at[g]], add=True)
        return 0
    lax.fori_loop(0, GPB, _group, 0)
    plsc.subcore_barrier()

    def _wout(j, _):
        pltpu.sync_copy(acc_sh.at[pl.ds(s * NPT + j * G, G)], rows_v)
        pltpu.sync_copy(rows_v, out_hbm.at[c, pl.ds(s * NPT + j * G, G)])
        return 0
    lax.fori_loop(0, NPT // G, _wout, 0)


def _elu_body(p0_ref, p1_ref, o_ref):
    lo = p0_ref[...]
    hi = p1_ref[...]
    o_ref[:, :FH] = jnp.where(lo > 0, lo, jnp.exp(jnp.minimum(lo, 0.0)) - 1.0)
    o_ref[:, FH:] = jnp.where(hi > 0, hi, jnp.exp(jnp.minimum(hi, 0.0)) - 1.0)


def _elu_concat(p0, p1):
    blk = 1000
    return pl.pallas_call(
        _elu_body,
        grid=(N // blk,),
        in_specs=[
            pl.BlockSpec((blk, FH), lambda i: (i, 0)),
            pl.BlockSpec((blk, FH), lambda i: (i, 0)),
        ],
        out_specs=pl.BlockSpec((blk, F), lambda i: (i, 0)),
        out_shape=jax.ShapeDtypeStruct((N, F), jnp.float32),
    )(p0, p1)


def kernel(x, edge, W, attn):
    src = edge[0].astype(jnp.int32)
    dst = edge[1].astype(jnp.int32)
    attn_pad = jnp.concatenate(
        [attn[:F, None], attn[F:, None], jnp.zeros((F, F - 2), jnp.float32)],
        axis=1)
    h2, ha = _project(x.astype(jnp.float32), W.astype(jnp.float32), attn_pad)
    a_src = ha[:, 0]
    a_dst = ha[:, 1]
    ev, ssum_p = _attn_kernel(a_src, a_dst,
                              src.reshape(NW, GPT, G), dst.reshape(NW, GPT, G))
    out_p = _spmm_kernel(h2,
                         src.reshape(16, GPB, G), dst.reshape(16, GPB, G),
                         ev.reshape(16, GPB, G), _sum_partials(ssum_p))
    return _elu_concat(out_p[0, :N], out_p[1, :N])


# speedup vs baseline: 16.2325x; 1.1957x over previous
import functools

import jax
import jax.numpy as jnp
from jax import lax
from jax.experimental import pallas as pl
from jax.experimental.pallas import tpu as pltpu
from jax.experimental.pallas import tpu_sc as plsc

N = 10000
E = 320000
F = 128
ALPHA = 0.2
G = 80
EB = E // G
GPB = EB // 16
BG = 50
NB = GPB // BG
FH = F // 2
N_PAD = 10240
NPT = N_PAD // 16
NLD = N // 16

_sc_mesh = plsc.VectorSubcoreMesh(core_axis_name="c", subcore_axis_name="s")
_sc_params = pltpu.CompilerParams(needs_layout_passes=False,
                                  use_tc_tiling_on_sc=False)


def _proj_body(x_ref, wb_ref, w_ref, a_ref, h2_ref, ha_ref):
    h2_ref[...] = jnp.dot(x_ref[...], wb_ref[0],
                          preferred_element_type=jnp.float32)
    wa = jnp.dot(w_ref[...], a_ref[...], preferred_element_type=jnp.float32)
    ha_ref[...] = jnp.dot(x_ref[...], wa, preferred_element_type=jnp.float32)


def _project(x, W, attn_pad):
    blk = 1000
    nb = N // blk
    return pl.pallas_call(
        _proj_body,
        grid=(nb, 2),
        in_specs=[
            pl.BlockSpec((blk, F), lambda i, j: (i, 0)),
            pl.BlockSpec((1, F, FH), lambda i, j: (j, 0, 0)),
            pl.BlockSpec((F, F), lambda i, j: (0, 0)),
            pl.BlockSpec((F, F), lambda i, j: (0, 0)),
        ],
        out_specs=[
            pl.BlockSpec((blk, FH), lambda i, j: (j * nb + i, 0)),
            pl.BlockSpec((blk, F), lambda i, j: (i, 0)),
        ],
        out_shape=[
            jax.ShapeDtypeStruct((2 * N, FH), jnp.float32),
            jax.ShapeDtypeStruct((N, F), jnp.float32),
        ],
    )(x, W.reshape(F, 2, FH).transpose(1, 0, 2), W, attn_pad)


@functools.partial(
    pl.kernel,
    out_type=jax.ShapeDtypeStruct((2, N_PAD, FH), jnp.float32),
    mesh=_sc_mesh,
    scratch_types=[
        pltpu.VMEM((N,), jnp.float32),
        pltpu.VMEM((N,), jnp.float32),
        pltpu.VMEM((BG, G), jnp.int32),
        pltpu.VMEM((BG, G), jnp.int32),
        pltpu.VMEM((BG, G), jnp.float32),
        pltpu.VMEM((G, FH), jnp.float32),
        pltpu.VMEM((NPT,), jnp.float32),
        pltpu.VMEM_SHARED((N, FH), jnp.float32),
        pltpu.VMEM_SHARED((N_PAD, FH), jnp.float32),
        pltpu.VMEM_SHARED((N_PAD,), jnp.float32),
    ],
    compiler_params=_sc_params,
)
def _gat_kernel(asrc_hbm, adst_hbm, h2_hbm, src_hbm, dst_hbm, out_hbm,
                asrc_v, adst_v, src_v, dst_v, w_v, rows_v, sbuf,
                tab_sh, acc_sh, ssum_sh):
    c = lax.axis_index("c")
    s = lax.axis_index("s")
    pltpu.sync_copy(asrc_hbm, asrc_v)
    pltpu.sync_copy(adst_hbm, adst_v)
    pltpu.sync_copy(h2_hbm.at[pl.ds(c * N + s * NLD, NLD)],
                    tab_sh.at[pl.ds(s * NLD, NLD)])

    def _zrow(e, _):
        def _zc(k, _):
            rows_v[e, pl.ds(k * 16, 16)] = jnp.zeros((16,), jnp.float32)
            return 0
        lax.fori_loop(0, FH // 16, _zc, 0)
        return 0
    lax.fori_loop(0, G, _zrow, 0)

    def _zs(i, _):
        sbuf[pl.ds(i * 16, 16)] = jnp.zeros((16,), jnp.float32)
        return 0
    lax.fori_loop(0, NPT // 16, _zs, 0)
    pltpu.sync_copy(sbuf, ssum_sh.at[pl.ds(s * NPT, NPT)])

    def _zout(j, _):
        pltpu.sync_copy(rows_v, acc_sh.at[pl.ds(s * NPT + j * G, G)])
        return 0
    lax.fori_loop(0, NPT // G, _zout, 0)
    plsc.subcore_barrier()

    def _block(b, _):
        pltpu.sync_copy(src_hbm.at[s, pl.ds(b * BG, BG)], src_v)
        pltpu.sync_copy(dst_hbm.at[s, pl.ds(b * BG, BG)], dst_v)

        def _group(g, _):
            def _sub(k, _):
                si = src_v[g, pl.ds(k * 16, 16)]
                di = dst_v[g, pl.ds(k * 16, 16)]
                v = (plsc.load_gather(asrc_v, [si])
                     + plsc.load_gather(adst_v, [di]))
                val = jnp.where(v > 0, -v, (-ALPHA) * v)
                w_v[g, pl.ds(k * 16, 16)] = jnp.exp(val)
                return 0
            lax.fori_loop(0, G // 16, _sub, 0)
            pltpu.sync_copy(w_v.at[g], ssum_sh.at[src_v.at[g]], add=True)
            pltpu.sync_copy(tab_sh.at[dst_v.at[g]], rows_v)

            def _edge(e, _):
                gb = jnp.full((16,), g, jnp.int32)
                eb = jnp.full((16,), e, jnp.int32)
                wb = plsc.load_gather(w_v, [gb, eb])

                def _chunk(k, _):
                    rows_v[e, pl.ds(k * 16, 16)] = (
                        rows_v[e, pl.ds(k * 16, 16)] * wb)
                    return 0
                lax.fori_loop(0, FH // 16, _chunk, 0)
                return 0
            lax.fori_loop(0, G, _edge, 0)
            pltpu.sync_copy(rows_v, acc_sh.at[src_v.at[g]], add=True)
            return 0
        lax.fori_loop(0, BG, _group, 0)
        return 0
    lax.fori_loop(0, NB, _block, 0)
    plsc.subcore_barrier()

    pltpu.sync_copy(ssum_sh.at[pl.ds(s * NPT, NPT)], sbuf)

    def _wout(j, _):
        pltpu.sync_copy(acc_sh.at[pl.ds(s * NPT + j * G, G)], rows_v)

        def _nrow(e, _):
            ib = jnp.full((16,), j * G + e, jnp.int32)
            sv = plsc.load_gather(sbuf, [ib])
            recip = 1.0 / (sv + 1e-16)

            def _chunk(k, _):
                rows_v[e, pl.ds(k * 16, 16)] = (
                    rows_v[e, pl.ds(k * 16, 16)] * recip)
                return 0
            lax.fori_loop(0, FH // 16, _chunk, 0)
            return 0
        lax.fori_loop(0, G, _nrow, 0)
        pltpu.sync_copy(rows_v, out_hbm.at[c, pl.ds(s * NPT + j * G, G)])
        return 0
    lax.fori_loop(0, NPT // G, _wout, 0)


def _elu_body(p0_ref, p1_ref, o_ref):
    lo = p0_ref[...]
    hi = p1_ref[...]
    o_ref[:, :FH] = jnp.where(lo > 0, lo, jnp.exp(jnp.minimum(lo, 0.0)) - 1.0)
    o_ref[:, FH:] = jnp.where(hi > 0, hi, jnp.exp(jnp.minimum(hi, 0.0)) - 1.0)


def _elu_concat(p0, p1):
    blk = 1000
    return pl.pallas_call(
        _elu_body,
        grid=(N // blk,),
        in_specs=[
            pl.BlockSpec((blk, FH), lambda i: (i, 0)),
            pl.BlockSpec((blk, FH), lambda i: (i, 0)),
        ],
        out_specs=pl.BlockSpec((blk, F), lambda i: (i, 0)),
        out_shape=jax.ShapeDtypeStruct((N, F), jnp.float32),
    )(p0, p1)


def kernel(x, edge, W, attn):
    src = edge[0].astype(jnp.int32)
    dst = edge[1].astype(jnp.int32)
    attn_pad = jnp.concatenate(
        [attn[:F, None], attn[F:, None], jnp.zeros((F, F - 2), jnp.float32)],
        axis=1)
    h2, ha = _project(x.astype(jnp.float32), W.astype(jnp.float32), attn_pad)
    a_src = ha[:, 0]
    a_dst = ha[:, 1]
    out_p = _gat_kernel(a_src, a_dst, h2,
                        src.reshape(16, GPB, G), dst.reshape(16, GPB, G))
    return _elu_concat(out_p[0, :N], out_p[1, :N])


# speedup vs baseline: 23.7568x; 1.4635x over previous
import functools

import jax
import jax.numpy as jnp
from jax import lax
from jax.experimental import pallas as pl
from jax.experimental.pallas import tpu as pltpu
from jax.experimental.pallas import tpu_sc as plsc

N = 10000
E = 320000
F = 128
ALPHA = 0.2
G = 80
EB = E // G
GPB = EB // 16
BG = 10
BGE = BG * G
NBLK = GPB // BG
FH = F // 2
N_PAD = 10240
NPT = N_PAD // 16
NLD = N // 16
NRING = 3

_sc_mesh = plsc.VectorSubcoreMesh(core_axis_name="c", subcore_axis_name="s")
_sc_params = pltpu.CompilerParams(needs_layout_passes=False,
                                  use_tc_tiling_on_sc=False)


def _proj_body(x_ref, wb_ref, w_ref, a_ref, h2_ref, ha_ref):
    h2_ref[...] = jnp.dot(x_ref[...], wb_ref[0],
                          preferred_element_type=jnp.float32)
    wa = jnp.dot(w_ref[...], a_ref[...], preferred_element_type=jnp.float32)
    ha_ref[...] = jnp.dot(x_ref[...], wa, preferred_element_type=jnp.float32)


def _project(x, W, attn_pad):
    blk = 1000
    nb = N // blk
    return pl.pallas_call(
        _proj_body,
        grid=(nb, 2),
        in_specs=[
            pl.BlockSpec((blk, F), lambda i, j: (i, 0)),
            pl.BlockSpec((1, F, FH), lambda i, j: (j, 0, 0)),
            pl.BlockSpec((F, F), lambda i, j: (0, 0)),
            pl.BlockSpec((F, F), lambda i, j: (0, 0)),
        ],
        out_specs=[
            pl.BlockSpec((blk, FH), lambda i, j: (j * nb + i, 0)),
            pl.BlockSpec((blk, F), lambda i, j: (i, 0)),
        ],
        out_shape=[
            jax.ShapeDtypeStruct((2 * N, FH), jnp.float32),
            jax.ShapeDtypeStruct((N, F), jnp.float32),
        ],
    )(x, W.reshape(F, 2, FH).transpose(1, 0, 2), W, attn_pad)


@functools.partial(
    pl.kernel,
    out_type=jax.ShapeDtypeStruct((N, F), jnp.float32),
    mesh=_sc_mesh,
    scratch_types=[
        pltpu.VMEM((N,), jnp.float32),
        pltpu.VMEM((N,), jnp.float32),
        pltpu.VMEM((BGE,), jnp.int32),
        pltpu.VMEM((BGE,), jnp.int32),
        pltpu.VMEM((BGE,), jnp.float32),
        pltpu.VMEM((NRING, G, FH), jnp.float32),
        pltpu.VMEM((NPT,), jnp.float32),
        pltpu.VMEM_SHARED((N, FH), jnp.float32),
        pltpu.VMEM_SHARED((N_PAD, FH), jnp.float32),
        pltpu.VMEM_SHARED((N_PAD,), jnp.float32),
        pltpu.SemaphoreType.DMA,
        pltpu.SemaphoreType.DMA,
        pltpu.SemaphoreType.DMA,
        pltpu.SemaphoreType.DMA,
        pltpu.SemaphoreType.DMA,
        pltpu.SemaphoreType.DMA,
        pltpu.SemaphoreType.DMA,
    ],
    compiler_params=_sc_params,
)
def _gat_kernel(asrc_hbm, adst_hbm, h2_hbm, src_hbm, dst_hbm, out_hbm,
                asrc_v, adst_v, src_v, dst_v, w_v, ring_v, sbuf,
                tab_sh, acc_sh, ssum_sh,
                gs0, gs1, gs2, ss0, ss1, ss2, bsem):
    c = lax.axis_index("c")
    s = lax.axis_index("s")
    gsems = (gs0, gs1, gs2)
    ssems = (ss0, ss1, ss2)
    pltpu.sync_copy(asrc_hbm, asrc_v)
    pltpu.sync_copy(adst_hbm, adst_v)
    pltpu.sync_copy(h2_hbm.at[pl.ds(c * N + s * NLD, NLD)],
                    tab_sh.at[pl.ds(s * NLD, NLD)])

    def _zrow(e, _):
        def _zc(k, _):
            ring_v[0, e, pl.ds(k * 16, 16)] = jnp.zeros((16,), jnp.float32)
            return 0
        lax.fori_loop(0, FH // 16, _zc, 0)
        return 0
    lax.fori_loop(0, G, _zrow, 0)

    def _zs(i, _):
        sbuf[pl.ds(i * 16, 16)] = jnp.zeros((16,), jnp.float32)
        return 0
    lax.fori_loop(0, NPT // 16, _zs, 0)
    pltpu.sync_copy(sbuf, ssum_sh.at[pl.ds(s * NPT, NPT)])

    def _zout(j, _):
        pltpu.sync_copy(ring_v.at[0], acc_sh.at[pl.ds(s * NPT + j * G, G)])
        return 0
    lax.fori_loop(0, NPT // G, _zout, 0)
    plsc.subcore_barrier()

    def _mult(buf, base):
        def _edge(e, _):
            wb = plsc.load_gather(w_v, [jnp.full((16,), base + e, jnp.int32)])

            def _chunk(k, _):
                ring_v[buf, e, pl.ds(k * 16, 16)] = (
                    ring_v[buf, e, pl.ds(k * 16, 16)] * wb)
                return 0
            lax.fori_loop(0, FH // 16, _chunk, 0)
            return 0
        lax.fori_loop(0, G, _edge, 0)

    def _block(b, _):
        pltpu.sync_copy(src_hbm.at[s, b], src_v)
        pltpu.sync_copy(dst_hbm.at[s, b], dst_v)

        def _ev(k, _):
            si = src_v[pl.ds(k * 16, 16)]
            di = dst_v[pl.ds(k * 16, 16)]
            v = (plsc.load_gather(asrc_v, [si])
                 + plsc.load_gather(adst_v, [di]))
            w_v[pl.ds(k * 16, 16)] = jnp.exp(
                jnp.where(v > 0, -v, (-ALPHA) * v))
            return 0
        lax.fori_loop(0, BGE // 16, _ev, 0)

        hsum = [pltpu.async_copy(w_v.at[pl.ds(g * G, G)],
                                 ssum_sh.at[src_v.at[pl.ds(g * G, G)]],
                                 bsem, add=True)
                for g in range(BG)]

        gh = [None] * NRING
        sh = [None] * NRING
        for g in range(BG + 1):
            if g < BG:
                i = g % NRING
                if sh[i] is not None:
                    sh[i].wait()
                gh[i] = pltpu.async_copy(
                    tab_sh.at[dst_v.at[pl.ds(g * G, G)]],
                    ring_v.at[i], gsems[i])
            if g >= 1:
                j = (g - 1) % NRING
                gh[j].wait()
                _mult(j, (g - 1) * G)
                sh[j] = pltpu.async_copy(
                    ring_v.at[j],
                    acc_sh.at[src_v.at[pl.ds((g - 1) * G, G)]],
                    ssems[j], add=True)
        for h in sh:
            if h is not None:
                h.wait()
        for h in hsum:
            h.wait()
        return 0
    lax.fori_loop(0, NBLK, _block, 0)
    plsc.subcore_barrier()

    pltpu.sync_copy(ssum_sh.at[pl.ds(s * NPT, NPT)], sbuf)

    def _wout(j, _):
        @pl.when(s * NPT + j * G + G <= N)
        def _valid():
            pltpu.sync_copy(acc_sh.at[pl.ds(s * NPT + j * G, G)],
                            ring_v.at[0])

            def _nrow(e, _):
                ib = jnp.full((16,), j * G + e, jnp.int32)
                sv = plsc.load_gather(sbuf, [ib])
                recip = 1.0 / (sv + 1e-16)

                def _chunk(k, _):
                    val = ring_v[0, e, pl.ds(k * 16, 16)] * recip
                    ring_v[0, e, pl.ds(k * 16, 16)] = jnp.where(
                        val > 0, val, jnp.exp(jnp.minimum(val, 0.0)) - 1.0)
                    return 0
                lax.fori_loop(0, FH // 16, _chunk, 0)
                return 0
            lax.fori_loop(0, G, _nrow, 0)
            pltpu.sync_copy(ring_v.at[0],
                            out_hbm.at[pl.ds(s * NPT + j * G, G),
                                       pl.ds(c * FH, FH)])
        return 0
    lax.fori_loop(0, NPT // G, _wout, 0)


def kernel(x, edge, W, attn):
    src = edge[0].astype(jnp.int32)
    dst = edge[1].astype(jnp.int32)
    attn_pad = jnp.concatenate(
        [attn[:F, None], attn[F:, None], jnp.zeros((F, F - 2), jnp.float32)],
        axis=1)
    h2, ha = _project(x.astype(jnp.float32), W.astype(jnp.float32), attn_pad)
    return _gat_kernel(ha[:, 0], ha[:, 1], h2,
                       src.reshape(16, NBLK, BGE), dst.reshape(16, NBLK, BGE))


# speedup vs baseline: 27.9382x; 1.1760x over previous
import functools

import jax
import jax.numpy as jnp
from jax import lax
from jax.experimental import pallas as pl
from jax.experimental.pallas import tpu as pltpu
from jax.experimental.pallas import tpu_sc as plsc

N = 10000
E = 320000
F = 128
ALPHA = 0.2
G = 80
EB = E // G
GPB = EB // 16
BG = 10
BGE = BG * G
NBLK = GPB // BG
FH = F // 2
N_PAD = 10240
NPT = N_PAD // 16
NLD = N // 16
NRING = 3

_sc_mesh = plsc.VectorSubcoreMesh(core_axis_name="c", subcore_axis_name="s")
_sc_params = pltpu.CompilerParams(needs_layout_passes=False,
                                  use_tc_tiling_on_sc=False)


def _proj_body(x_ref, wb_ref, w_ref, a_ref, h2_ref, ha_ref):
    h2_ref[...] = jnp.dot(x_ref[...], wb_ref[0],
                          preferred_element_type=jnp.float32)
    wa = jnp.dot(w_ref[...], a_ref[...], preferred_element_type=jnp.float32)
    ha_ref[...] = jnp.dot(x_ref[...], wa, preferred_element_type=jnp.float32)


def _project(x, W, attn_pad):
    blk = 1000
    nb = N // blk
    return pl.pallas_call(
        _proj_body,
        grid=(nb, 2),
        in_specs=[
            pl.BlockSpec((blk, F), lambda i, j: (i, 0)),
            pl.BlockSpec((1, F, FH), lambda i, j: (j, 0, 0)),
            pl.BlockSpec((F, F), lambda i, j: (0, 0)),
            pl.BlockSpec((F, F), lambda i, j: (0, 0)),
        ],
        out_specs=[
            pl.BlockSpec((blk, FH), lambda i, j: (j * nb + i, 0)),
            pl.BlockSpec((blk, F), lambda i, j: (i, 0)),
        ],
        out_shape=[
            jax.ShapeDtypeStruct((2 * N, FH), jnp.float32),
            jax.ShapeDtypeStruct((N, F), jnp.float32),
        ],
    )(x, W.reshape(F, 2, FH).transpose(1, 0, 2), W, attn_pad)


@functools.partial(
    pl.kernel,
    out_type=jax.ShapeDtypeStruct((N, F), jnp.float32),
    mesh=_sc_mesh,
    scratch_types=[
        pltpu.VMEM((N,), jnp.float32),
        pltpu.VMEM((N,), jnp.float32),
        pltpu.VMEM((BGE,), jnp.int32),
        pltpu.VMEM((BGE,), jnp.int32),
        pltpu.VMEM((BGE,), jnp.float32),
        pltpu.VMEM((NRING, G, FH), jnp.float32),
        pltpu.VMEM((NPT,), jnp.float32),
        pltpu.VMEM_SHARED((N, FH), jnp.float32),
        pltpu.VMEM_SHARED((N_PAD, FH), jnp.float32),
        pltpu.VMEM_SHARED((N_PAD,), jnp.float32),
        pltpu.SemaphoreType.DMA,
        pltpu.SemaphoreType.DMA,
        pltpu.SemaphoreType.DMA,
        pltpu.SemaphoreType.DMA,
        pltpu.SemaphoreType.DMA,
        pltpu.SemaphoreType.DMA,
        pltpu.SemaphoreType.DMA,
    ],
    compiler_params=_sc_params,
)
def _gat_kernel(asrc_hbm, adst_hbm, h2_hbm, src_hbm, dst_hbm, out_hbm,
                asrc_v, adst_v, src_v, dst_v, w_v, ring_v, sbuf,
                tab_sh, acc_sh, ssum_sh,
                gs0, gs1, gs2, ss0, ss1, ss2, bsem):
    c = lax.axis_index("c")
    s = lax.axis_index("s")
    gsems = (gs0, gs1, gs2)
    ssems = (ss0, ss1, ss2)
    pltpu.sync_copy(asrc_hbm, asrc_v)
    pltpu.sync_copy(adst_hbm, adst_v)
    pltpu.sync_copy(h2_hbm.at[pl.ds(c * N + s * NLD, NLD)],
                    tab_sh.at[pl.ds(s * NLD, NLD)])

    def _zrow(e, _):
        def _zc(k, _):
            ring_v[0, e, pl.ds(k * 16, 16)] = jnp.zeros((16,), jnp.float32)
            return 0
        lax.fori_loop(0, FH // 16, _zc, 0)
        return 0
    lax.fori_loop(0, G, _zrow, 0)

    def _zs(i, _):
        sbuf[pl.ds(i * 16, 16)] = jnp.zeros((16,), jnp.float32)
        return 0
    lax.fori_loop(0, NPT // 16, _zs, 0)
    pltpu.sync_copy(sbuf, ssum_sh.at[pl.ds(s * NPT, NPT)])

    def _zout(j, _):
        pltpu.sync_copy(ring_v.at[0], acc_sh.at[pl.ds(s * NPT + j * G, G)])
        return 0
    lax.fori_loop(0, NPT // G, _zout, 0)
    plsc.subcore_barrier()

    def _mult(buf, base):
        @plsc.parallel_loop(0, G, step=1, unroll=4)
        def _edge(e):
            wb = plsc.load_gather(w_v, [jnp.full((16,), base + e, jnp.int32)])
            for k in range(FH // 16):
                ring_v[buf, e, pl.ds(k * 16, 16)] = (
                    ring_v[buf, e, pl.ds(k * 16, 16)] * wb)

    def _block(b, _):
        pltpu.sync_copy(src_hbm.at[s, b], src_v)
        pltpu.sync_copy(dst_hbm.at[s, b], dst_v)

        @plsc.parallel_loop(0, BGE, step=16, unroll=4)
        def _ev(i):
            si = src_v[pl.ds(i, 16)]
            di = dst_v[pl.ds(i, 16)]
            v = (plsc.load_gather(asrc_v, [si])
                 + plsc.load_gather(adst_v, [di]))
            w_v[pl.ds(i, 16)] = jnp.exp(
                jnp.where(v > 0, -v, (-ALPHA) * v))

        hsum = [pltpu.async_copy(w_v.at[pl.ds(g * G, G)],
                                 ssum_sh.at[src_v.at[pl.ds(g * G, G)]],
                                 bsem, add=True)
                for g in range(BG)]

        gh = [None] * NRING
        sh = [None] * NRING
        for g in range(BG + 1):
            if g < BG:
                i = g % NRING
                if sh[i] is not None:
                    sh[i].wait()
                gh[i] = pltpu.async_copy(
                    tab_sh.at[dst_v.at[pl.ds(g * G, G)]],
                    ring_v.at[i], gsems[i])
            if g >= 1:
                j = (g - 1) % NRING
                gh[j].wait()
                _mult(j, (g - 1) * G)
                sh[j] = pltpu.async_copy(
                    ring_v.at[j],
                    acc_sh.at[src_v.at[pl.ds((g - 1) * G, G)]],
                    ssems[j], add=True)
        for h in sh:
            if h is not None:
                h.wait()
        for h in hsum:
            h.wait()
        return 0
    lax.fori_loop(0, NBLK, _block, 0)
    plsc.subcore_barrier()

    pltpu.sync_copy(ssum_sh.at[pl.ds(s * NPT, NPT)], sbuf)

    def _wout(j, _):
        @pl.when(s * NPT + j * G + G <= N)
        def _valid():
            pltpu.sync_copy(acc_sh.at[pl.ds(s * NPT + j * G, G)],
                            ring_v.at[0])

            @plsc.parallel_loop(0, G, step=1, unroll=4)
            def _nrow(e):
                ib = jnp.full((16,), j * G + e, jnp.int32)
                sv = plsc.load_gather(sbuf, [ib])
                recip = 1.0 / (sv + 1e-16)
                for k in range(FH // 16):
                    val = ring_v[0, e, pl.ds(k * 16, 16)] * recip
                    ring_v[0, e, pl.ds(k * 16, 16)] = jnp.where(
                        val > 0, val, jnp.exp(jnp.minimum(val, 0.0)) - 1.0)
            pltpu.sync_copy(ring_v.at[0],
                            out_hbm.at[pl.ds(s * NPT + j * G, G),
                                       pl.ds(c * FH, FH)])
        return 0
    lax.fori_loop(0, NPT // G, _wout, 0)


def kernel(x, edge, W, attn):
    src = edge[0].astype(jnp.int32)
    dst = edge[1].astype(jnp.int32)
    attn_pad = jnp.concatenate(
        [attn[:F, None], attn[F:, None], jnp.zeros((F, F - 2), jnp.float32)],
        axis=1)
    h2, ha = _project(x.astype(jnp.float32), W.astype(jnp.float32), attn_pad)
    return _gat_kernel(ha[:, 0], ha[:, 1], h2,
                       src.reshape(16, NBLK, BGE), dst.reshape(16, NBLK, BGE))


# speedup vs baseline: 29.8542x; 1.0686x over previous
import functools

import jax
import jax.numpy as jnp
from jax import lax
from jax.experimental import pallas as pl
from jax.experimental.pallas import tpu as pltpu
from jax.experimental.pallas import tpu_sc as plsc

N = 10000
E = 320000
F = 128
ALPHA = 0.2
G = 80
EB = E // G
GPB = EB // 16
BG = 25
BGE = BG * G
NBLK = GPB // BG
FH = F // 2
N_PAD = 10240
NPT = N_PAD // 16
NLD = N // 16
NRING = 3

_sc_mesh = plsc.VectorSubcoreMesh(core_axis_name="c", subcore_axis_name="s")
_sc_params = pltpu.CompilerParams(needs_layout_passes=False,
                                  use_tc_tiling_on_sc=False)


def _proj_body(x_ref, wb_ref, w_ref, a_ref, h2_ref, ha_ref):
    h2_ref[...] = jnp.dot(x_ref[...], wb_ref[0],
                          preferred_element_type=jnp.float32)
    wa = jnp.dot(w_ref[...], a_ref[...], preferred_element_type=jnp.float32)
    ha_ref[...] = jnp.dot(x_ref[...], wa, preferred_element_type=jnp.float32)


def _project(x, W, attn_pad):
    blk = 1000
    nb = N // blk
    return pl.pallas_call(
        _proj_body,
        grid=(nb, 2),
        in_specs=[
            pl.BlockSpec((blk, F), lambda i, j: (i, 0)),
            pl.BlockSpec((1, F, FH), lambda i, j: (j, 0, 0)),
            pl.BlockSpec((F, F), lambda i, j: (0, 0)),
            pl.BlockSpec((F, F), lambda i, j: (0, 0)),
        ],
        out_specs=[
            pl.BlockSpec((blk, FH), lambda i, j: (j * nb + i, 0)),
            pl.BlockSpec((blk, F), lambda i, j: (i, 0)),
        ],
        out_shape=[
            jax.ShapeDtypeStruct((2 * N, FH), jnp.float32),
            jax.ShapeDtypeStruct((N, F), jnp.float32),
        ],
    )(x, W.reshape(F, 2, FH).transpose(1, 0, 2), W, attn_pad)


@functools.partial(
    pl.kernel,
    out_type=jax.ShapeDtypeStruct((N, F), jnp.float32),
    mesh=_sc_mesh,
    scratch_types=[
        pltpu.VMEM((N,), jnp.float32),
        pltpu.VMEM((N,), jnp.float32),
        pltpu.VMEM((BGE,), jnp.int32),
        pltpu.VMEM((BGE,), jnp.int32),
        pltpu.VMEM((BGE,), jnp.float32),
        pltpu.VMEM((NRING, G, FH), jnp.float32),
        pltpu.VMEM((NPT,), jnp.float32),
        pltpu.VMEM_SHARED((N, FH), jnp.float32),
        pltpu.VMEM_SHARED((N_PAD, FH), jnp.float32),
        pltpu.VMEM_SHARED((N_PAD,), jnp.float32),
        pltpu.SemaphoreType.DMA,
        pltpu.SemaphoreType.DMA,
        pltpu.SemaphoreType.DMA,
        pltpu.SemaphoreType.DMA,
        pltpu.SemaphoreType.DMA,
        pltpu.SemaphoreType.DMA,
        pltpu.SemaphoreType.DMA,
    ],
    compiler_params=_sc_params,
)
def _gat_kernel(asrc_hbm, adst_hbm, h2_hbm, src_hbm, dst_hbm, out_hbm,
                asrc_v, adst_v, src_v, dst_v, w_v, ring_v, sbuf,
                tab_sh, acc_sh, ssum_sh,
                gs0, gs1, gs2, ss0, ss1, ss2, bsem):
    c = lax.axis_index("c")
    s = lax.axis_index("s")
    gsems = (gs0, gs1, gs2)
    ssems = (ss0, ss1, ss2)
    pltpu.sync_copy(asrc_hbm, asrc_v)
    pltpu.sync_copy(adst_hbm, adst_v)
    pltpu.sync_copy(h2_hbm.at[pl.ds(c * N + s * NLD, NLD)],
                    tab_sh.at[pl.ds(s * NLD, NLD)])

    def _zrow(e, _):
        def _zc(k, _):
            ring_v[0, e, pl.ds(k * 16, 16)] = jnp.zeros((16,), jnp.float32)
            return 0
        lax.fori_loop(0, FH // 16, _zc, 0)
        return 0
    lax.fori_loop(0, G, _zrow, 0)

    def _zs(i, _):
        sbuf[pl.ds(i * 16, 16)] = jnp.zeros((16,), jnp.float32)
        return 0
    lax.fori_loop(0, NPT // 16, _zs, 0)
    pltpu.sync_copy(sbuf, ssum_sh.at[pl.ds(s * NPT, NPT)])

    def _zout(j, _):
        pltpu.sync_copy(ring_v.at[0], acc_sh.at[pl.ds(s * NPT + j * G, G)])
        return 0
    lax.fori_loop(0, NPT // G, _zout, 0)
    plsc.subcore_barrier()

    def _mult(buf, base):
        @plsc.parallel_loop(0, G, step=1, unroll=8)
        def _edge(e):
            wb = plsc.load_gather(w_v, [jnp.full((16,), base + e, jnp.int32)])
            for k in range(FH // 16):
                ring_v[buf, e, pl.ds(k * 16, 16)] = (
                    ring_v[buf, e, pl.ds(k * 16, 16)] * wb)

    def _block(b, _):
        pltpu.sync_copy(src_hbm.at[s, b], src_v)
        pltpu.sync_copy(dst_hbm.at[s, b], dst_v)

        @plsc.parallel_loop(0, BGE, step=16, unroll=4)
        def _ev(i):
            si = src_v[pl.ds(i, 16)]
            di = dst_v[pl.ds(i, 16)]
            v = (plsc.load_gather(asrc_v, [si])
                 + plsc.load_gather(adst_v, [di]))
            w_v[pl.ds(i, 16)] = jnp.exp(
                jnp.where(v > 0, -v, (-ALPHA) * v))

        hsum = [pltpu.async_copy(w_v.at[pl.ds(g * G, G)],
                                 ssum_sh.at[src_v.at[pl.ds(g * G, G)]],
                                 bsem, add=True)
                for g in range(BG)]

        gh = [None] * NRING
        sh = [None] * NRING
        for g in range(BG + 1):
            if g < BG:
                i = g % NRING
                if sh[i] is not None:
                    sh[i].wait()
                gh[i] = pltpu.async_copy(
                    tab_sh.at[dst_v.at[pl.ds(g * G, G)]],
                    ring_v.at[i], gsems[i])
            if g >= 1:
                j = (g - 1) % NRING
                gh[j].wait()
                _mult(j, (g - 1) * G)
                sh[j] = pltpu.async_copy(
                    ring_v.at[j],
                    acc_sh.at[src_v.at[pl.ds((g - 1) * G, G)]],
                    ssems[j], add=True)
        for h in sh:
            if h is not None:
                h.wait()
        for h in hsum:
            h.wait()
        return 0
    lax.fori_loop(0, NBLK, _block, 0)
    plsc.subcore_barrier()

    pltpu.sync_copy(ssum_sh.at[pl.ds(s * NPT, NPT)], sbuf)

    def _wout(j, _):
        @pl.when(s * NPT + j * G + G <= N)
        def _valid():
            pltpu.sync_copy(acc_sh.at[pl.ds(s * NPT + j * G, G)],
                            ring_v.at[0])

            @plsc.parallel_loop(0, G, step=1, unroll=4)
            def _nrow(e):
                ib = jnp.full((16,), j * G + e, jnp.int32)
                sv = plsc.load_gather(sbuf, [ib])
                recip = 1.0 / (sv + 1e-16)
                for k in range(FH // 16):
                    val = ring_v[0, e, pl.ds(k * 16, 16)] * recip
                    ring_v[0, e, pl.ds(k * 16, 16)] = jnp.where(
                        val > 0, val, jnp.exp(jnp.minimum(val, 0.0)) - 1.0)
            pltpu.sync_copy(ring_v.at[0],
                            out_hbm.at[pl.ds(s * NPT + j * G, G),
                                       pl.ds(c * FH, FH)])
        return 0
    lax.fori_loop(0, NPT // G, _wout, 0)


def kernel(x, edge, W, attn):
    src = edge[0].astype(jnp.int32)
    dst = edge[1].astype(jnp.int32)
    attn_pad = jnp.concatenate(
        [attn[:F, None], attn[F:, None], jnp.zeros((F, F - 2), jnp.float32)],
        axis=1)
    h2, ha = _project(x.astype(jnp.float32), W.astype(jnp.float32), attn_pad)
    return _gat_kernel(ha[:, 0], ha[:, 1], h2,
                       src.reshape(16, NBLK, BGE), dst.reshape(16, NBLK, BGE))


# speedup vs baseline: 29.8811x; 1.0009x over previous
import functools

import jax
import jax.numpy as jnp
from jax import lax
from jax.experimental import pallas as pl
from jax.experimental.pallas import tpu as pltpu
from jax.experimental.pallas import tpu_sc as plsc

N = 10000
E = 320000
F = 128
ALPHA = 0.2
G = 80
EB = E // G
GPB = EB // 16
BG = 25
BGE = BG * G
NBLK = GPB // BG
FH = F // 2
N_PAD = 10240
NPT = N_PAD // 16
NLD = N // 16
NRING = 3

_sc_mesh = plsc.VectorSubcoreMesh(core_axis_name="c", subcore_axis_name="s")
_sc_params = pltpu.CompilerParams(needs_layout_passes=False,
                                  use_tc_tiling_on_sc=False)


def _proj_body(x_ref, wb_ref, w_ref, a_ref, h2_ref, ha_ref):
    h2_ref[...] = jnp.dot(x_ref[...], wb_ref[0],
                          preferred_element_type=jnp.float32)
    wa = jnp.dot(w_ref[...], a_ref[...], preferred_element_type=jnp.float32)
    ha_ref[...] = jnp.dot(x_ref[...], wa, preferred_element_type=jnp.float32)


def _project(x, W, attn_pad):
    blk = 1000
    nb = N // blk
    return pl.pallas_call(
        _proj_body,
        grid=(nb, 2),
        in_specs=[
            pl.BlockSpec((blk, F), lambda i, j: (i, 0)),
            pl.BlockSpec((1, F, FH), lambda i, j: (j, 0, 0)),
            pl.BlockSpec((F, F), lambda i, j: (0, 0)),
            pl.BlockSpec((F, F), lambda i, j: (0, 0)),
        ],
        out_specs=[
            pl.BlockSpec((blk, FH), lambda i, j: (j * nb + i, 0)),
            pl.BlockSpec((blk, F), lambda i, j: (i, 0)),
        ],
        out_shape=[
            jax.ShapeDtypeStruct((2 * N, FH), jnp.float32),
            jax.ShapeDtypeStruct((N, F), jnp.float32),
        ],
    )(x, W.reshape(F, 2, FH).transpose(1, 0, 2), W, attn_pad)


@functools.partial(
    pl.kernel,
    out_type=jax.ShapeDtypeStruct((N, F), jnp.float32),
    mesh=_sc_mesh,
    scratch_types=[
        pltpu.VMEM((N,), jnp.float32),
        pltpu.VMEM((N,), jnp.float32),
        pltpu.VMEM((BGE,), jnp.int32),
        pltpu.VMEM((BGE,), jnp.int32),
        pltpu.VMEM((BGE,), jnp.float32),
        pltpu.VMEM((NRING, G, FH), jnp.float32),
        pltpu.VMEM((NPT,), jnp.float32),
        pltpu.VMEM_SHARED((N, FH), jnp.float32),
        pltpu.VMEM_SHARED((N_PAD, FH), jnp.float32),
        pltpu.VMEM_SHARED((N_PAD,), jnp.float32),
        pltpu.SemaphoreType.DMA,
        pltpu.SemaphoreType.DMA,
        pltpu.SemaphoreType.DMA,
        pltpu.SemaphoreType.DMA,
        pltpu.SemaphoreType.DMA,
        pltpu.SemaphoreType.DMA,
        pltpu.SemaphoreType.DMA,
    ],
    compiler_params=_sc_params,
)
def _gat_kernel(asrc_hbm, adst_hbm, h2_hbm, src_hbm, dst_hbm, out_hbm,
                asrc_v, adst_v, src_v, dst_v, w_v, ring_v, sbuf,
                tab_sh, acc_sh, ssum_sh,
                gs0, gs1, gs2, ss0, ss1, ss2, bsem):
    c = lax.axis_index("c")
    s = lax.axis_index("s")
    gsems = (gs0, gs1, gs2)
    ssems = (ss0, ss1, ss2)
    pltpu.sync_copy(asrc_hbm, asrc_v)
    pltpu.sync_copy(adst_hbm, adst_v)
    pltpu.sync_copy(h2_hbm.at[pl.ds(c * N + s * NLD, NLD)],
                    tab_sh.at[pl.ds(s * NLD, NLD)])

    def _zrow(e, _):
        def _zc(k, _):
            ring_v[0, e, pl.ds(k * 16, 16)] = jnp.zeros((16,), jnp.float32)
            return 0
        lax.fori_loop(0, FH // 16, _zc, 0)
        return 0
    lax.fori_loop(0, G, _zrow, 0)

    def _zs(i, _):
        sbuf[pl.ds(i * 16, 16)] = jnp.zeros((16,), jnp.float32)
        return 0
    lax.fori_loop(0, NPT // 16, _zs, 0)
    pltpu.sync_copy(sbuf, ssum_sh.at[pl.ds(s * NPT, NPT)])

    def _zout(j, _):
        pltpu.sync_copy(ring_v.at[0], acc_sh.at[pl.ds(s * NPT + j * G, G)])
        return 0
    lax.fori_loop(0, NPT // G, _zout, 0)
    plsc.subcore_barrier()

    def _mult(buf, base):
        @plsc.parallel_loop(0, G, step=16, unroll=2)
        def _e16(m):
            w16 = w_v[pl.ds(base + m, 16)]
            for e in range(16):
                wb = w16.at[jnp.full((16,), e, jnp.int32)].get(
                    mode="promise_in_bounds")
                for k in range(FH // 16):
                    ring_v[buf, m + e, pl.ds(k * 16, 16)] = (
                        ring_v[buf, m + e, pl.ds(k * 16, 16)] * wb)

    def _block(b, _):
        pltpu.sync_copy(src_hbm.at[s, b], src_v)
        pltpu.sync_copy(dst_hbm.at[s, b], dst_v)

        @plsc.parallel_loop(0, BGE, step=16, unroll=4)
        def _ev(i):
            si = src_v[pl.ds(i, 16)]
            di = dst_v[pl.ds(i, 16)]
            v = (plsc.load_gather(asrc_v, [si])
                 + plsc.load_gather(adst_v, [di]))
            w_v[pl.ds(i, 16)] = jnp.exp(
                jnp.where(v > 0, -v, (-ALPHA) * v))

        hsum = [pltpu.async_copy(w_v.at[pl.ds(g * G, G)],
                                 ssum_sh.at[src_v.at[pl.ds(g * G, G)]],
                                 bsem, add=True)
                for g in range(BG)]

        gh = [None] * NRING
        sh = [None] * NRING
        for g in range(BG + 1):
            if g < BG:
                i = g % NRING
                if sh[i] is not None:
                    sh[i].wait()
                gh[i] = pltpu.async_copy(
                    tab_sh.at[dst_v.at[pl.ds(g * G, G)]],
                    ring_v.at[i], gsems[i])
            if g >= 1:
                j = (g - 1) % NRING
                gh[j].wait()
                _mult(j, (g - 1) * G)
                sh[j] = pltpu.async_copy(
                    ring_v.at[j],
                    acc_sh.at[src_v.at[pl.ds((g - 1) * G, G)]],
                    ssems[j], add=True)
        for h in sh:
            if h is not None:
                h.wait()
        for h in hsum:
            h.wait()
        return 0
    lax.fori_loop(0, NBLK, _block, 0)
    plsc.subcore_barrier()

    pltpu.sync_copy(ssum_sh.at[pl.ds(s * NPT, NPT)], sbuf)

    def _wout(j, _):
        @pl.when(s * NPT + j * G + G <= N)
        def _valid():
            pltpu.sync_copy(acc_sh.at[pl.ds(s * NPT + j * G, G)],
                            ring_v.at[0])

            @plsc.parallel_loop(0, G, step=1, unroll=4)
            def _nrow(e):
                ib = jnp.full((16,), j * G + e, jnp.int32)
                sv = plsc.load_gather(sbuf, [ib])
                recip = 1.0 / (sv + 1e-16)
                for k in range(FH // 16):
                    val = ring_v[0, e, pl.ds(k * 16, 16)] * recip
                    ring_v[0, e, pl.ds(k * 16, 16)] = jnp.where(
                        val > 0, val, jnp.exp(jnp.minimum(val, 0.0)) - 1.0)
            pltpu.sync_copy(ring_v.at[0],
                            out_hbm.at[pl.ds(s * NPT + j * G, G),
                                       pl.ds(c * FH, FH)])
        return 0
    lax.fori_loop(0, NPT // G, _wout, 0)


def kernel(x, edge, W, attn):
    src = edge[0].astype(jnp.int32)
    dst = edge[1].astype(jnp.int32)
    attn_pad = jnp.concatenate(
        [attn[:F, None], attn[F:, None], jnp.zeros((F, F - 2), jnp.float32)],
        axis=1)
    h2, ha = _project(x.astype(jnp.float32), W.astype(jnp.float32), attn_pad)
    return _gat_kernel(ha[:, 0], ha[:, 1], h2,
                       src.reshape(16, NBLK, BGE), dst.reshape(16, NBLK, BGE))
